# Initial kernel scaffold; baseline (speedup 1.0000x reference)
import jax
import jax.numpy as jnp
from jax.experimental import pallas as pl


def kernel(X, centers):
    raise NotImplementedError("write your pallas kernel here")


# speedup vs baseline: 1.2012x; 1.2012x over previous
import jax
import jax.numpy as jnp
from jax.experimental import pallas as pl

_BN = 256


def _assign_kernel(x_ref, c_ref, out_ref):
    x = x_ref[...]
    c = c_ref[...]
    k = c.shape[0]
    h = k // 2
    x2 = jnp.sum(x * x, axis=1, keepdims=True)
    c2 = jnp.sum(c * c, axis=1)[None, :]
    dot = jnp.dot(x, c.T, preferred_element_type=jnp.float32)
    d2 = jnp.maximum((x2 + c2) - 2.0 * dot, 0.0)

    da = d2[:, :h]
    db = d2[:, h:]
    iota = jax.lax.broadcasted_iota(jnp.int32, da.shape, 1)
    ma = jnp.min(da, axis=1, keepdims=True)
    ia = jnp.min(jnp.where(da == ma, iota, jnp.int32(h)), axis=1, keepdims=True)
    mb = jnp.min(db, axis=1, keepdims=True)
    ib = jnp.min(jnp.where(db == mb, iota, jnp.int32(h)), axis=1, keepdims=True)

    dist_a = jnp.sqrt(ma)
    dist_b = jnp.sqrt(mb)
    dist_a_r = dist_a.astype(jnp.bfloat16).astype(jnp.float32)
    pick_a = dist_a_r <= dist_b
    out_ref[...] = jnp.where(pick_a, ia, ib + jnp.int32(h))


def kernel(X, centers):
    n, d = X.shape
    k, _ = centers.shape
    grid = (n // _BN,)
    out = pl.pallas_call(
        _assign_kernel,
        grid=grid,
        in_specs=[
            pl.BlockSpec((_BN, d), lambda i: (i, 0)),
            pl.BlockSpec((k, d), lambda i: (0, 0)),
        ],
        out_specs=pl.BlockSpec((_BN, 1), lambda i: (i, 0)),
        out_shape=jax.ShapeDtypeStruct((n, 1), jnp.int32),
    )(X, centers)
    return out.reshape(n)


# speedup vs baseline: 1.3197x; 1.0986x over previous
import jax
import jax.numpy as jnp
from jax.experimental import pallas as pl

_BN = 256


def _assign_kernel(x_ref, x2_ref, c_ref, c2_ref, out_ref):
    x = x_ref[...]
    c = c_ref[...]
    x2 = x2_ref[...]
    c2 = c2_ref[...]
    k = c.shape[0]
    h = k // 2
    dot = jnp.dot(x, c.T, preferred_element_type=jnp.float32)
    d2 = (x2 + c2) - 2.0 * dot

    da = d2[:, :h]
    db = d2[:, h:]
    iota = jax.lax.broadcasted_iota(jnp.int32, da.shape, 1)
    ma = jnp.min(da, axis=1, keepdims=True)
    ia = jnp.min(jnp.where(da == ma, iota, jnp.int32(h)), axis=1, keepdims=True)
    mb = jnp.min(db, axis=1, keepdims=True)
    ib = jnp.min(jnp.where(db == mb, iota, jnp.int32(h)), axis=1, keepdims=True)

    dist_a = jnp.sqrt(jnp.maximum(ma, 0.0))
    dist_b = jnp.sqrt(jnp.maximum(mb, 0.0))
    dist_a_r = dist_a.astype(jnp.bfloat16).astype(jnp.float32)
    pick_a = dist_a_r <= dist_b
    out_ref[...] = jnp.where(pick_a, ia, ib + jnp.int32(h))


def kernel(X, centers):
    n, d = X.shape
    k, _ = centers.shape
    x2 = jnp.sum(X * X, axis=1, keepdims=True)
    c2 = jnp.sum(centers * centers, axis=1)[None, :]
    xb = X.astype(jnp.bfloat16)
    cb = centers.astype(jnp.bfloat16)
    grid = (n // _BN,)
    out = pl.pallas_call(
        _assign_kernel,
        grid=grid,
        in_specs=[
            pl.BlockSpec((_BN, d), lambda i: (i, 0)),
            pl.BlockSpec((_BN, 1), lambda i: (i, 0)),
            pl.BlockSpec((k, d), lambda i: (0, 0)),
            pl.BlockSpec((1, k), lambda i: (0, 0)),
        ],
        out_specs=pl.BlockSpec((_BN, 1), lambda i: (i, 0)),
        out_shape=jax.ShapeDtypeStruct((n, 1), jnp.int32),
    )(xb, x2, cb, c2)
    return out.reshape(n)


# speedup vs baseline: 1.4078x; 1.0668x over previous
import jax
import jax.numpy as jnp
from jax.experimental import pallas as pl

_BN = 512


def _assign_kernel(x_ref, x2_ref, c_ref, c2_ref, out_ref):
    x = x_ref[...]
    c = c_ref[...]
    x2 = x2_ref[...]
    c2 = c2_ref[...]
    k = c.shape[0]
    h = k // 2
    dot = jnp.dot(x, c.T, preferred_element_type=jnp.float32)
    d2 = (x2 + c2) - 2.0 * dot

    da = d2[:, :h]
    db = d2[:, h:]
    iota = jax.lax.broadcasted_iota(jnp.int32, da.shape, 1)
    ma = jnp.min(da, axis=1, keepdims=True)
    ia = jnp.min(jnp.where(da == ma, iota, jnp.int32(h)), axis=1, keepdims=True)
    mb = jnp.min(db, axis=1, keepdims=True)
    ib = jnp.min(jnp.where(db == mb, iota, jnp.int32(h)), axis=1, keepdims=True)

    dist_a = jnp.sqrt(jnp.maximum(ma, 0.0))
    dist_b = jnp.sqrt(jnp.maximum(mb, 0.0))
    dist_a_r = dist_a.astype(jnp.bfloat16).astype(jnp.float32)
    pick_a = dist_a_r <= dist_b
    out_ref[...] = jnp.where(pick_a, ia, ib + jnp.int32(h))


def kernel(X, centers):
    n, d = X.shape
    k, _ = centers.shape
    x2 = jnp.sum(X * X, axis=1, keepdims=True)
    c2 = jnp.sum(centers * centers, axis=1)[None, :]
    xb = X.astype(jnp.bfloat16)
    cb = centers.astype(jnp.bfloat16)
    grid = (n // _BN,)
    out = pl.pallas_call(
        _assign_kernel,
        grid=grid,
        in_specs=[
            pl.BlockSpec((_BN, d), lambda i: (i, 0)),
            pl.BlockSpec((_BN, 1), lambda i: (i, 0)),
            pl.BlockSpec((k, d), lambda i: (0, 0)),
            pl.BlockSpec((1, k), lambda i: (0, 0)),
        ],
        out_specs=pl.BlockSpec((_BN, 1), lambda i: (i, 0)),
        out_shape=jax.ShapeDtypeStruct((n, 1), jnp.int32),
    )(xb, x2, cb, c2)
    return out.reshape(n)


# speedup vs baseline: 1.5198x; 1.0795x over previous
import jax
import jax.numpy as jnp
from jax.experimental import pallas as pl

_BN = 1024


def _assign_kernel(x_ref, x2_ref, c_ref, c2_ref, out_ref):
    x = x_ref[...]
    c = c_ref[...]
    x2 = x2_ref[...]
    c2 = c2_ref[...]
    k = c.shape[0]
    h = k // 2
    dot = jnp.dot(x, c.T, preferred_element_type=jnp.float32)
    d2 = (x2 + c2) - 2.0 * dot

    da = d2[:, :h]
    db = d2[:, h:]
    iota = jax.lax.broadcasted_iota(jnp.int32, da.shape, 1)
    ma = jnp.min(da, axis=1, keepdims=True)
    ia = jnp.min(jnp.where(da == ma, iota, jnp.int32(h)), axis=1, keepdims=True)
    mb = jnp.min(db, axis=1, keepdims=True)
    ib = jnp.min(jnp.where(db == mb, iota, jnp.int32(h)), axis=1, keepdims=True)

    dist_a = jnp.sqrt(jnp.maximum(ma, 0.0))
    dist_b = jnp.sqrt(jnp.maximum(mb, 0.0))
    dist_a_r = dist_a.astype(jnp.bfloat16).astype(jnp.float32)
    pick_a = dist_a_r <= dist_b
    out_ref[...] = jnp.where(pick_a, ia, ib + jnp.int32(h))


def kernel(X, centers):
    n, d = X.shape
    k, _ = centers.shape
    x2 = jnp.sum(X * X, axis=1, keepdims=True)
    c2 = jnp.sum(centers * centers, axis=1)[None, :]
    xb = X.astype(jnp.bfloat16)
    cb = centers.astype(jnp.bfloat16)
    grid = (n // _BN,)
    out = pl.pallas_call(
        _assign_kernel,
        grid=grid,
        in_specs=[
            pl.BlockSpec((_BN, d), lambda i: (i, 0)),
            pl.BlockSpec((_BN, 1), lambda i: (i, 0)),
            pl.BlockSpec((k, d), lambda i: (0, 0)),
            pl.BlockSpec((1, k), lambda i: (0, 0)),
        ],
        out_specs=pl.BlockSpec((_BN, 1), lambda i: (i, 0)),
        out_shape=jax.ShapeDtypeStruct((n, 1), jnp.int32),
    )(xb, x2, cb, c2)
    return out.reshape(n)


# speedup vs baseline: 1.5766x; 1.0374x over previous
import jax
import jax.numpy as jnp
from jax.experimental import pallas as pl

_BN = 2048


def _assign_kernel(x_ref, x2_ref, c_ref, c2_ref, out_ref):
    x = x_ref[...]
    c = c_ref[...]
    x2 = x2_ref[...]
    c2 = c2_ref[...]
    k = c.shape[0]
    h = k // 2
    dot = jnp.dot(x, c.T, preferred_element_type=jnp.float32)
    d2 = (x2 + c2) - 2.0 * dot

    da = d2[:, :h]
    db = d2[:, h:]
    iota = jax.lax.broadcasted_iota(jnp.int32, da.shape, 1)
    ma = jnp.min(da, axis=1, keepdims=True)
    ia = jnp.min(jnp.where(da == ma, iota, jnp.int32(h)), axis=1, keepdims=True)
    mb = jnp.min(db, axis=1, keepdims=True)
    ib = jnp.min(jnp.where(db == mb, iota, jnp.int32(h)), axis=1, keepdims=True)

    dist_a = jnp.sqrt(jnp.maximum(ma, 0.0))
    dist_b = jnp.sqrt(jnp.maximum(mb, 0.0))
    dist_a_r = dist_a.astype(jnp.bfloat16).astype(jnp.float32)
    pick_a = dist_a_r <= dist_b
    out_ref[...] = jnp.where(pick_a, ia, ib + jnp.int32(h))


def kernel(X, centers):
    n, d = X.shape
    k, _ = centers.shape
    x2 = jnp.sum(X * X, axis=1, keepdims=True)
    c2 = jnp.sum(centers * centers, axis=1)[None, :]
    xb = X.astype(jnp.bfloat16)
    cb = centers.astype(jnp.bfloat16)
    grid = (n // _BN,)
    out = pl.pallas_call(
        _assign_kernel,
        grid=grid,
        in_specs=[
            pl.BlockSpec((_BN, d), lambda i: (i, 0)),
            pl.BlockSpec((_BN, 1), lambda i: (i, 0)),
            pl.BlockSpec((k, d), lambda i: (0, 0)),
            pl.BlockSpec((1, k), lambda i: (0, 0)),
        ],
        out_specs=pl.BlockSpec((_BN, 1), lambda i: (i, 0)),
        out_shape=jax.ShapeDtypeStruct((n, 1), jnp.int32),
    )(xb, x2, cb, c2)
    return out.reshape(n)
